# Initial kernel scaffold; baseline (speedup 1.0000x reference)
import jax
import jax.numpy as jnp
from jax.experimental import pallas as pl


def kernel(node_feature, edge_feature, edge_index, W, attn_l, attn_r, attn_e):
    raise NotImplementedError("write your pallas kernel here")


# speedup vs baseline: 11.8945x; 11.8945x over previous
import functools

import jax
import jax.numpy as jnp
from jax import lax
from jax.experimental import pallas as pl
from jax.experimental.pallas import tpu as pltpu
from jax.experimental.pallas import tpu_sc as plsc

N = 10000
E = 320000
IN_DIM = 128
H = 4
D = 32
HD = H * D
ALPHA = 0.2

NP = 10240
NTILES = 32
EPT = E // NTILES
CH = 80
NCH = EPT // CH
EPT5 = E // 16
NCH5 = EPT5 // CH
VPC = CH * H // 16
ROWS_PER_TILE = NP // 16

_NEG_INF = float("-inf")


def _leaky(x):
    return jnp.where(x > 0, x, ALPHA * x)


def _padmax(a):
    m128 = jnp.concatenate(
        [jnp.max(a, axis=0), jnp.full((128 - H,), _NEG_INF, jnp.float32)])
    rows = lax.broadcasted_iota(jnp.int32, (8, 128), 0)
    return jnp.where(rows == 0, m128[None, :], _NEG_INF)


def _k1_body(x_ref, w_ref, sl_ref, sr_ref, ft_ref, a1_ref, a2_ref, g1_ref):
    i = pl.program_id(0)
    ft = jnp.dot(x_ref[...], w_ref[...], preferred_element_type=jnp.float32)
    ft_ref[...] = ft
    a1 = jnp.dot(ft, sl_ref[...], preferred_element_type=jnp.float32)
    a2 = jnp.dot(ft, sr_ref[...], preferred_element_type=jnp.float32)
    a1_ref[...] = a1
    a2_ref[...] = a2

    @pl.when(i == 0)
    def _():
        g1_ref[...] = jnp.full((8, 128), _NEG_INF, jnp.float32)

    g1_ref[...] = jnp.maximum(g1_ref[...], _padmax(a1))


def _run_k1(x_pad, w, s_l, s_r):
    blk = 2048
    grid = NP // blk
    return pl.pallas_call(
        _k1_body,
        grid=(grid,),
        in_specs=[
            pl.BlockSpec((blk, IN_DIM), lambda i: (i, 0)),
            pl.BlockSpec((IN_DIM, HD), lambda i: (0, 0)),
            pl.BlockSpec((HD, H), lambda i: (0, 0)),
            pl.BlockSpec((HD, H), lambda i: (0, 0)),
        ],
        out_specs=[
            pl.BlockSpec((blk, HD), lambda i: (i, 0)),
            pl.BlockSpec((blk, H), lambda i: (i, 0)),
            pl.BlockSpec((blk, H), lambda i: (i, 0)),
            pl.BlockSpec((8, 128), lambda i: (0, 0)),
        ],
        out_shape=[
            jax.ShapeDtypeStruct((NP, HD), jnp.float32),
            jax.ShapeDtypeStruct((NP, H), jnp.float32),
            jax.ShapeDtypeStruct((NP, H), jnp.float32),
            jax.ShapeDtypeStruct((8, 128), jnp.float32),
        ],
    )(x_pad, w, s_l, s_r)


def _k2_body(x_ref, w_ref, se_ref, eft_ref, a3_ref, g3_ref):
    i = pl.program_id(0)
    eft = jnp.dot(x_ref[...], w_ref[...], preferred_element_type=jnp.float32)
    eft_ref[...] = eft
    a3 = jnp.dot(eft, se_ref[...], preferred_element_type=jnp.float32)
    a3_ref[...] = a3

    @pl.when(i == 0)
    def _():
        g3_ref[...] = jnp.full((8, 128), _NEG_INF, jnp.float32)

    g3_ref[...] = jnp.maximum(g3_ref[...], _padmax(a3))


def _run_k2(edge_feature, w, s_e):
    blk = 2000
    grid = E // blk
    return pl.pallas_call(
        _k2_body,
        grid=(grid,),
        in_specs=[
            pl.BlockSpec((blk, IN_DIM), lambda i: (i, 0)),
            pl.BlockSpec((IN_DIM, HD), lambda i: (0, 0)),
            pl.BlockSpec((HD, H), lambda i: (0, 0)),
        ],
        out_specs=[
            pl.BlockSpec((blk, HD), lambda i: (i, 0)),
            pl.BlockSpec((blk, H), lambda i: (i, 0)),
            pl.BlockSpec((8, 128), lambda i: (0, 0)),
        ],
        out_shape=[
            jax.ShapeDtypeStruct((E, HD), jnp.float32),
            jax.ShapeDtypeStruct((E, H), jnp.float32),
            jax.ShapeDtypeStruct((8, 128), jnp.float32),
        ],
    )(edge_feature, w, s_e)


def _k3_body(a1_hbm, a2_hbm, a3_hbm, srcf_hbm, dstf_hbm, g1_hbm, g3_hbm,
             z4_hbm, expa_hbm, p_hbm,
             acc_v, a1_v, a2_v, src_c, dst_c, a3c_v, g1_v, g3_v, expa_v):
    cid = lax.axis_index("c")
    sid = lax.axis_index("s")
    wid = sid * 2 + cid

    pltpu.sync_copy(a1_hbm, a1_v)
    pltpu.sync_copy(a2_hbm, a2_v)
    pltpu.sync_copy(g1_hbm, g1_v)
    pltpu.sync_copy(g3_hbm, g3_v)
    pltpu.sync_copy(z4_hbm, acc_v)

    iota = lax.iota(jnp.int32, 16)
    lane_h = jnp.bitwise_and(iota, 3)
    lane_lo = iota < H
    g13 = (plsc.load_gather(g1_v, [lane_h]) +
           plsc.load_gather(g3_v, [lane_h]))

    def _chunk(c, _):
        base = wid * EPT + c * CH
        pltpu.sync_copy(srcf_hbm.at[pl.ds(base, CH)], src_c)
        pltpu.sync_copy(dstf_hbm.at[pl.ds(base, CH)], dst_c)
        pltpu.sync_copy(a3_hbm.at[pl.ds(base * H, CH * H)], a3c_v)

        def _edge(e, _):
            e16 = jnp.full((16,), e, jnp.int32)
            ssp = plsc.load_gather(src_c, [e16])
            dsp = plsc.load_gather(dst_c, [e16])
            eh = e16 * H + lane_h
            v1 = plsc.load_gather(a1_v, [ssp * H + lane_h])
            didx = dsp * H + lane_h
            v2 = plsc.load_gather(a2_v, [didx])
            v3 = plsc.load_gather(a3c_v, [eh])
            a = _leaky(v1 + v3 + v2)
            u = _leaky(g13 + v2)
            ev = jnp.where(lane_lo, jnp.exp(a - u), 0.0)
            cur = plsc.load_gather(acc_v, [didx])
            plsc.store_scatter(acc_v, [didx], cur + ev, mask=lane_lo)
            plsc.store_scatter(expa_v, [eh], ev, mask=lane_lo)
            return ()

        lax.fori_loop(0, CH, _edge, ())
        pltpu.sync_copy(expa_v, expa_hbm.at[pl.ds(base * H, CH * H)])
        return ()

    lax.fori_loop(0, NCH, _chunk, ())
    pltpu.sync_copy(acc_v, p_hbm.at[wid])


def _run_k3(a1f, a2f, a3f, srcf, dstf, g1v, g3v, z4):
    mesh = plsc.VectorSubcoreMesh(core_axis_name="c", subcore_axis_name="s")
    kern = functools.partial(
        pl.kernel,
        mesh=mesh,
        out_type=[
            jax.ShapeDtypeStruct((E * H,), jnp.float32),
            jax.ShapeDtypeStruct((NTILES, NP * H), jnp.float32),
        ],
        scratch_types=[
            pltpu.VMEM((NP * H,), jnp.float32),
            pltpu.VMEM((NP * H,), jnp.float32),
            pltpu.VMEM((NP * H,), jnp.float32),
            pltpu.VMEM((CH,), jnp.int32),
            pltpu.VMEM((CH,), jnp.int32),
            pltpu.VMEM((CH * H,), jnp.float32),
            pltpu.VMEM((16,), jnp.float32),
            pltpu.VMEM((16,), jnp.float32),
            pltpu.VMEM((CH * H,), jnp.float32),
        ],
        compiler_params=pltpu.CompilerParams(needs_layout_passes=False),
    )(_k3_body)
    return kern(a1f, a2f, a3f, srcf, dstf, g1v, g3v, z4)


def _k4_body(p_ref, recip_ref):
    recip_ref[...] = 1.0 / jnp.sum(p_ref[...], axis=0, keepdims=True)


def _run_k4(p):
    return pl.pallas_call(
        _k4_body,
        grid=(1,),
        in_specs=[pl.BlockSpec((NTILES, NP * H), lambda i: (0, 0))],
        out_specs=pl.BlockSpec((1, NP * H), lambda i: (0, 0)),
        out_shape=jax.ShapeDtypeStruct((1, NP * H), jnp.float32),
    )(p)


def _k4b_body(expa_hbm, recip_hbm, dstf_hbm, adrop_hbm,
              recip_v, dst_c, expa_v, adrop_v):
    cid = lax.axis_index("c")
    sid = lax.axis_index("s")
    wid = sid * 2 + cid

    pltpu.sync_copy(recip_hbm, recip_v)
    iota = lax.iota(jnp.int32, 16)
    lane_h = jnp.bitwise_and(iota, 3)
    lane_e = lax.shift_right_logical(iota, 2)

    def _chunk(c, _):
        base = wid * EPT + c * CH
        pltpu.sync_copy(dstf_hbm.at[pl.ds(base, CH)], dst_c)
        pltpu.sync_copy(expa_hbm.at[pl.ds(base * H, CH * H)], expa_v)
        for v in range(VPC):
            lidx = jnp.full((16,), v * 4, jnp.int32) + lane_e
            dstl = plsc.load_gather(dst_c, [lidx])
            rl = plsc.load_gather(recip_v, [dstl * H + lane_h])
            ev = expa_v[pl.ds(v * 16, 16)]
            adrop_v[pl.ds(v * 16, 16)] = ev * rl
        pltpu.sync_copy(adrop_v, adrop_hbm.at[pl.ds(base * H, CH * H)])
        return ()

    lax.fori_loop(0, NCH, _chunk, ())


def _run_k4b(expa_flat, recipf, dstf):
    mesh = plsc.VectorSubcoreMesh(core_axis_name="c", subcore_axis_name="s")
    kern = functools.partial(
        pl.kernel,
        mesh=mesh,
        out_type=jax.ShapeDtypeStruct((E * H,), jnp.float32),
        scratch_types=[
            pltpu.VMEM((NP * H,), jnp.float32),
            pltpu.VMEM((CH,), jnp.int32),
            pltpu.VMEM((CH * H,), jnp.float32),
            pltpu.VMEM((CH * H,), jnp.float32),
        ],
        compiler_params=pltpu.CompilerParams(needs_layout_passes=False),
    )(_k4b_body)
    return kern(expa_flat, recipf, dstf)

HHD = HD // 2


def _k5_body(ft_hbm, eft_hbm, adropf_hbm, srcf_hbm,
             dst3_hbm, ret2_hbm, fhi_hbm, p2lo_hbm, p2hi_hbm,
             src_c, dst2_v, ftr_v, eft_v, flo_v, fhi2_v,
             fflat_v, adrop_v, acc_sp, sem):
    cid = lax.axis_index("c")
    sid = lax.axis_index("s")
    wid = sid * 2 + cid

    pltpu.sync_copy(dst3_hbm.at[wid], dst2_v)

    zero16 = jnp.zeros((16,), jnp.float32)

    def _zbuf(e, _):
        for q in range(HHD // 16):
            fhi2_v[e, pl.ds(q * 16, 16)] = zero16
        return ()

    def _zero_table():
        lax.fori_loop(0, CH, _zbuf, ())

        def _zstripe(j, _):
            pltpu.sync_copy(
                fhi2_v, acc_sp.at[pl.ds(sid * ROWS_PER_TILE + j * CH, CH)])
            return ()

        lax.fori_loop(0, ROWS_PER_TILE // CH, _zstripe, ())

    def _dump_table(dst_flat_hbm):
        def _piece(j, _):
            row0 = sid * ROWS_PER_TILE + j * CH
            pltpu.sync_copy(acc_sp.at[pl.ds(row0, CH)], fhi2_v)

            def _row(e, _):
                for q in range(HHD // 16):
                    fflat_v[pl.ds(e * HHD + q * 16, 16)] = (
                        fhi2_v[e, pl.ds(q * 16, 16)])
                return ()

            lax.fori_loop(0, CH, _row, ())
            pltpu.sync_copy(fflat_v,
                            dst_flat_hbm.at[pl.ds(row0 * HHD, CH * HHD)])
            return ()

        lax.fori_loop(0, ROWS_PER_TILE // CH, _piece, ())

    _zero_table()
    plsc.subcore_barrier()

    def _chunk_a(c, _):
        erow = wid * EPT + c * CH
        pltpu.sync_copy(srcf_hbm.at[pl.ds(erow, CH)], src_c)
        pltpu.async_copy(ft_hbm.at[src_c], ftr_v, sem).wait()
        pltpu.sync_copy(eft_hbm.at[pl.ds(erow, CH)], eft_v)
        pltpu.sync_copy(adropf_hbm.at[pl.ds(erow * H, CH * H)], adrop_v)

        def _edge(e, _):
            for h in range(H):
                sp = plsc.load_gather(adrop_v, [jnp.full((16,), e * H + h,
                                                         jnp.int32)])
                for q in range(2):
                    col = h * D + q * 16
                    f = (ftr_v[e, pl.ds(col, 16)] +
                         eft_v[e, pl.ds(col, 16)]) * sp
                    if col < HHD:
                        flo_v[e, pl.ds(col, 16)] = f
                    else:
                        fflat_v[pl.ds(e * HHD + col - HHD, 16)] = f
                    eft_v[e, pl.ds(col, 16)] = jnp.where(
                        f > 0, f, jnp.exp(f) - 1.0)
            return ()

        lax.fori_loop(0, CH, _edge, ())
        pltpu.sync_copy(flo_v, acc_sp.at[dst2_v.at[c]], add=True)
        pltpu.sync_copy(fflat_v, fhi_hbm.at[pl.ds(erow * HHD, CH * HHD)])
        pltpu.sync_copy(eft_v, ret2_hbm.at[pl.ds(erow, CH)])
        return ()

    lax.fori_loop(0, NCH, _chunk_a, ())
    plsc.subcore_barrier()
    _dump_table(p2lo_hbm.at[cid])
    plsc.subcore_barrier()
    _zero_table()
    plsc.subcore_barrier()

    def _chunk_b(c, _):
        erow = wid * EPT + c * CH
        pltpu.sync_copy(fhi_hbm.at[pl.ds(erow * HHD, CH * HHD)], fflat_v)

        def _row(e, _):
            for q in range(HHD // 16):
                fhi2_v[e, pl.ds(q * 16, 16)] = (
                    fflat_v[pl.ds(e * HHD + q * 16, 16)])
            return ()

        lax.fori_loop(0, CH, _row, ())
        pltpu.sync_copy(fhi2_v, acc_sp.at[dst2_v.at[c]], add=True)
        return ()

    lax.fori_loop(0, NCH, _chunk_b, ())
    plsc.subcore_barrier()
    _dump_table(p2hi_hbm.at[cid])


def _run_k5(ft, eft, adrop_flat, srcf, dst3):
    mesh = plsc.VectorSubcoreMesh(core_axis_name="c", subcore_axis_name="s")
    kern = functools.partial(
        pl.kernel,
        mesh=mesh,
        out_type=[
            jax.ShapeDtypeStruct((E, HD), jnp.float32),
            jax.ShapeDtypeStruct((E * HHD,), jnp.float32),
            jax.ShapeDtypeStruct((2, NP * HHD), jnp.float32),
            jax.ShapeDtypeStruct((2, NP * HHD), jnp.float32),
        ],
        scratch_types=[
            pltpu.VMEM((CH,), jnp.int32),
            pltpu.VMEM((NCH, CH), jnp.int32),
            pltpu.VMEM((CH, HD), jnp.float32),
            pltpu.VMEM((CH, HD), jnp.float32),
            pltpu.VMEM((CH, HHD), jnp.float32),
            pltpu.VMEM((CH, HHD), jnp.float32),
            pltpu.VMEM((CH * HHD,), jnp.float32),
            pltpu.VMEM((CH * H,), jnp.float32),
            pltpu.VMEM_SHARED((NP, HHD), jnp.float32),
            pltpu.SemaphoreType.DMA,
        ],
        compiler_params=pltpu.CompilerParams(needs_layout_passes=False),
    )(_k5_body)
    return kern(ft, eft, adrop_flat, srcf, dst3)


def _k6_body(plo_ref, phi_ref, out_ref):
    s = jnp.concatenate([plo_ref[0] + plo_ref[1], phi_ref[0] + phi_ref[1]],
                        axis=1)
    out_ref[...] = jnp.where(s > 0, s, jnp.exp(s) - 1.0)


def _run_k6(p2lo, p2hi):
    blk = 2048
    return pl.pallas_call(
        _k6_body,
        grid=(NP // blk,),
        in_specs=[pl.BlockSpec((2, blk, HHD), lambda i: (0, i, 0)),
                  pl.BlockSpec((2, blk, HHD), lambda i: (0, i, 0))],
        out_specs=pl.BlockSpec((blk, HD), lambda i: (i, 0)),
        out_shape=jax.ShapeDtypeStruct((NP, HD), jnp.float32),
    )(p2lo, p2hi)


def kernel(node_feature, edge_feature, edge_index, W, attn_l, attn_r, attn_e):
    f32 = jnp.float32
    eye = jnp.eye(H, dtype=f32)
    s_l = (attn_l[..., 0][:, :, None] * eye[:, None, :]).reshape(HD, H)
    s_r = (attn_r[..., 0][:, :, None] * eye[:, None, :]).reshape(HD, H)
    s_e = (attn_e[..., 0][:, :, None] * eye[:, None, :]).reshape(HD, H)

    x_pad = jnp.pad(node_feature, ((0, NP - N), (0, 0)))

    ft, a1, a2, g1 = _run_k1(x_pad, W, s_l, s_r)
    eft, a3, g3 = _run_k2(edge_feature, W, s_e)

    g1v = g1.reshape(-1)[:16]
    g3v = g3.reshape(-1)[:16]
    srcf = edge_index[0].reshape(-1)
    dstf = edge_index[1].reshape(-1)
    dst3 = edge_index[1].reshape(NTILES, NCH, CH)
    z4 = jnp.zeros((NP * H,), f32)

    expa_flat, p_asum = _run_k3(a1.reshape(-1), a2.reshape(-1),
                                a3.reshape(-1), srcf, dstf, g1v, g3v, z4)
    recip = _run_k4(p_asum)
    adrop_flat = _run_k4b(expa_flat, recip.reshape(-1), dstf)
    adrop = jnp.repeat(adrop_flat.reshape(E, H), D, axis=1)
    f_cat = (ft[:N][srcf] + eft) * adrop
    ret2 = jnp.where(f_cat > 0, f_cat, jnp.expm1(f_cat))
    node_ft = jax.ops.segment_sum(f_cat, dstf, num_segments=N)
    ret1 = jnp.where(node_ft > 0, node_ft, jnp.expm1(node_ft))
    return ret1, ret2


# speedup vs baseline: 13.4098x; 1.1274x over previous
import functools

import jax
import jax.numpy as jnp
from jax import lax
from jax.experimental import pallas as pl
from jax.experimental.pallas import tpu as pltpu
from jax.experimental.pallas import tpu_sc as plsc

N = 10000
E = 320000
IN_DIM = 128
H = 4
D = 32
HD = H * D
ALPHA = 0.2

NP = 10240
NTILES = 32
EPT = E // NTILES
CH = 80
NCH = EPT // CH
EPT5 = E // 16
NCH5 = EPT5 // CH
VPC = CH * H // 16
ROWS_PER_TILE = NP // 16

_NEG_INF = float("-inf")


def _leaky(x):
    return jnp.where(x > 0, x, ALPHA * x)


def _padmax(a):
    m128 = jnp.concatenate(
        [jnp.max(a, axis=0), jnp.full((128 - H,), _NEG_INF, jnp.float32)])
    rows = lax.broadcasted_iota(jnp.int32, (8, 128), 0)
    return jnp.where(rows == 0, m128[None, :], _NEG_INF)


def _k1_body(x_ref, w_ref, sl_ref, sr_ref, ft_ref, a1_ref, a2_ref, g1_ref):
    i = pl.program_id(0)
    ft = jnp.dot(x_ref[...], w_ref[...], preferred_element_type=jnp.float32)
    ft_ref[...] = ft
    a1 = jnp.dot(ft, sl_ref[...], preferred_element_type=jnp.float32)
    a2 = jnp.dot(ft, sr_ref[...], preferred_element_type=jnp.float32)
    a1_ref[...] = a1
    a2_ref[...] = a2

    @pl.when(i == 0)
    def _():
        g1_ref[...] = jnp.full((8, 128), _NEG_INF, jnp.float32)

    g1_ref[...] = jnp.maximum(g1_ref[...], _padmax(a1))


def _run_k1(x_pad, w, s_l, s_r):
    blk = 2048
    grid = NP // blk
    return pl.pallas_call(
        _k1_body,
        grid=(grid,),
        in_specs=[
            pl.BlockSpec((blk, IN_DIM), lambda i: (i, 0)),
            pl.BlockSpec((IN_DIM, HD), lambda i: (0, 0)),
            pl.BlockSpec((HD, H), lambda i: (0, 0)),
            pl.BlockSpec((HD, H), lambda i: (0, 0)),
        ],
        out_specs=[
            pl.BlockSpec((blk, HD), lambda i: (i, 0)),
            pl.BlockSpec((blk, H), lambda i: (i, 0)),
            pl.BlockSpec((blk, H), lambda i: (i, 0)),
            pl.BlockSpec((8, 128), lambda i: (0, 0)),
        ],
        out_shape=[
            jax.ShapeDtypeStruct((NP, HD), jnp.float32),
            jax.ShapeDtypeStruct((NP, H), jnp.float32),
            jax.ShapeDtypeStruct((NP, H), jnp.float32),
            jax.ShapeDtypeStruct((8, 128), jnp.float32),
        ],
    )(x_pad, w, s_l, s_r)


def _k2_body(x_ref, w_ref, se_ref, eft_ref, a3_ref, g3_ref):
    i = pl.program_id(0)
    eft = jnp.dot(x_ref[...], w_ref[...], preferred_element_type=jnp.float32)
    eft_ref[...] = eft
    a3 = jnp.dot(eft, se_ref[...], preferred_element_type=jnp.float32)
    a3_ref[...] = a3

    @pl.when(i == 0)
    def _():
        g3_ref[...] = jnp.full((8, 128), _NEG_INF, jnp.float32)

    g3_ref[...] = jnp.maximum(g3_ref[...], _padmax(a3))


def _run_k2(edge_feature, w, s_e):
    blk = 2000
    grid = E // blk
    return pl.pallas_call(
        _k2_body,
        grid=(grid,),
        in_specs=[
            pl.BlockSpec((blk, IN_DIM), lambda i: (i, 0)),
            pl.BlockSpec((IN_DIM, HD), lambda i: (0, 0)),
            pl.BlockSpec((HD, H), lambda i: (0, 0)),
        ],
        out_specs=[
            pl.BlockSpec((blk, HD), lambda i: (i, 0)),
            pl.BlockSpec((blk, H), lambda i: (i, 0)),
            pl.BlockSpec((8, 128), lambda i: (0, 0)),
        ],
        out_shape=[
            jax.ShapeDtypeStruct((E, HD), jnp.float32),
            jax.ShapeDtypeStruct((E, H), jnp.float32),
            jax.ShapeDtypeStruct((8, 128), jnp.float32),
        ],
    )(edge_feature, w, s_e)


def _k3_body(a1_hbm, a2_hbm, a3_hbm, srcf_hbm, dstf_hbm, g1_hbm, g3_hbm,
             z4_hbm, expa_hbm, p_hbm,
             acc_v, a1_v, a2_v, src_c, dst_c, a3c_v, g1_v, g3_v, expa_v):
    cid = lax.axis_index("c")
    sid = lax.axis_index("s")
    wid = sid * 2 + cid

    pltpu.sync_copy(a1_hbm, a1_v)
    pltpu.sync_copy(a2_hbm, a2_v)
    pltpu.sync_copy(g1_hbm, g1_v)
    pltpu.sync_copy(g3_hbm, g3_v)
    pltpu.sync_copy(z4_hbm, acc_v)

    iota = lax.iota(jnp.int32, 16)
    lane_h = jnp.bitwise_and(iota, 3)
    lane_lo = iota < H
    g13 = (plsc.load_gather(g1_v, [lane_h]) +
           plsc.load_gather(g3_v, [lane_h]))

    def _chunk(c, _):
        base = wid * EPT + c * CH
        pltpu.sync_copy(srcf_hbm.at[pl.ds(base, CH)], src_c)
        pltpu.sync_copy(dstf_hbm.at[pl.ds(base, CH)], dst_c)
        pltpu.sync_copy(a3_hbm.at[pl.ds(base * H, CH * H)], a3c_v)

        def _edge(e, _):
            e16 = jnp.full((16,), e, jnp.int32)
            ssp = plsc.load_gather(src_c, [e16])
            dsp = plsc.load_gather(dst_c, [e16])
            eh = e16 * H + lane_h
            v1 = plsc.load_gather(a1_v, [ssp * H + lane_h])
            didx = dsp * H + lane_h
            v2 = plsc.load_gather(a2_v, [didx])
            v3 = plsc.load_gather(a3c_v, [eh])
            a = _leaky(v1 + v3 + v2)
            u = _leaky(g13 + v2)
            ev = jnp.where(lane_lo, jnp.exp(a - u), 0.0)
            cur = plsc.load_gather(acc_v, [didx])
            plsc.store_scatter(acc_v, [didx], cur + ev, mask=lane_lo)
            plsc.store_scatter(expa_v, [eh], ev, mask=lane_lo)
            return ()

        lax.fori_loop(0, CH, _edge, ())
        pltpu.sync_copy(expa_v, expa_hbm.at[pl.ds(base * H, CH * H)])
        return ()

    lax.fori_loop(0, NCH, _chunk, ())
    pltpu.sync_copy(acc_v, p_hbm.at[wid])


def _run_k3(a1f, a2f, a3f, srcf, dstf, g1v, g3v, z4):
    mesh = plsc.VectorSubcoreMesh(core_axis_name="c", subcore_axis_name="s")
    kern = functools.partial(
        pl.kernel,
        mesh=mesh,
        out_type=[
            jax.ShapeDtypeStruct((E * H,), jnp.float32),
            jax.ShapeDtypeStruct((NTILES, NP * H), jnp.float32),
        ],
        scratch_types=[
            pltpu.VMEM((NP * H,), jnp.float32),
            pltpu.VMEM((NP * H,), jnp.float32),
            pltpu.VMEM((NP * H,), jnp.float32),
            pltpu.VMEM((CH,), jnp.int32),
            pltpu.VMEM((CH,), jnp.int32),
            pltpu.VMEM((CH * H,), jnp.float32),
            pltpu.VMEM((16,), jnp.float32),
            pltpu.VMEM((16,), jnp.float32),
            pltpu.VMEM((CH * H,), jnp.float32),
        ],
        compiler_params=pltpu.CompilerParams(needs_layout_passes=False),
    )(_k3_body)
    return kern(a1f, a2f, a3f, srcf, dstf, g1v, g3v, z4)


def _k4_body(p_ref, recip_ref):
    recip_ref[...] = 1.0 / jnp.sum(p_ref[...], axis=0, keepdims=True)


def _run_k4(p):
    return pl.pallas_call(
        _k4_body,
        grid=(1,),
        in_specs=[pl.BlockSpec((NTILES, NP * H), lambda i: (0, 0))],
        out_specs=pl.BlockSpec((1, NP * H), lambda i: (0, 0)),
        out_shape=jax.ShapeDtypeStruct((1, NP * H), jnp.float32),
    )(p)


def _k4b_body(expa_hbm, recip_hbm, dstf_hbm, adrop_hbm,
              recip_v, dst_c, expa_v, adrop_v):
    cid = lax.axis_index("c")
    sid = lax.axis_index("s")
    wid = sid * 2 + cid

    pltpu.sync_copy(recip_hbm, recip_v)
    iota = lax.iota(jnp.int32, 16)
    lane_h = jnp.bitwise_and(iota, 3)
    lane_e = lax.shift_right_logical(iota, 2)

    def _chunk(c, _):
        base = wid * EPT + c * CH
        pltpu.sync_copy(dstf_hbm.at[pl.ds(base, CH)], dst_c)
        pltpu.sync_copy(expa_hbm.at[pl.ds(base * H, CH * H)], expa_v)
        for v in range(VPC):
            lidx = jnp.full((16,), v * 4, jnp.int32) + lane_e
            dstl = plsc.load_gather(dst_c, [lidx])
            rl = plsc.load_gather(recip_v, [dstl * H + lane_h])
            ev = expa_v[pl.ds(v * 16, 16)]
            adrop_v[pl.ds(v * 16, 16)] = ev * rl
        pltpu.sync_copy(adrop_v, adrop_hbm.at[pl.ds(base * H, CH * H)])
        return ()

    lax.fori_loop(0, NCH, _chunk, ())


def _run_k4b(expa_flat, recipf, dstf):
    mesh = plsc.VectorSubcoreMesh(core_axis_name="c", subcore_axis_name="s")
    kern = functools.partial(
        pl.kernel,
        mesh=mesh,
        out_type=jax.ShapeDtypeStruct((E * H,), jnp.float32),
        scratch_types=[
            pltpu.VMEM((NP * H,), jnp.float32),
            pltpu.VMEM((CH,), jnp.int32),
            pltpu.VMEM((CH * H,), jnp.float32),
            pltpu.VMEM((CH * H,), jnp.float32),
        ],
        compiler_params=pltpu.CompilerParams(needs_layout_passes=False),
    )(_k4b_body)
    return kern(expa_flat, recipf, dstf)

HHD = HD // 2


def _k5_body(ft_hbm, eftf_hbm, adropf_hbm, srcf_hbm, dstf_hbm,
             ret2f_hbm, fhi_hbm, p2lo_hbm, p2hi_hbm,
             src_c, dst2_v, ftr_v, eftf_v, flo_v, fhi2_v,
             fflat_v, adrop_v, acc_sp, sem):
    cid = lax.axis_index("c")
    sid = lax.axis_index("s")
    wid = sid * 2 + cid

    zero16 = jnp.zeros((16,), jnp.float32)

    def _zbuf(e, _):
        for q in range(HHD // 16):
            fhi2_v[e, pl.ds(q * 16, 16)] = zero16
        return ()

    def _zero_table():
        lax.fori_loop(0, CH, _zbuf, ())

        def _zstripe(j, _):
            pltpu.sync_copy(
                fhi2_v, acc_sp.at[pl.ds(sid * ROWS_PER_TILE + j * CH, CH)])
            return ()

        lax.fori_loop(0, ROWS_PER_TILE // CH, _zstripe, ())

    def _dump_table(dst_flat_hbm):
        def _piece(j, _):
            row0 = sid * ROWS_PER_TILE + j * CH
            pltpu.sync_copy(acc_sp.at[pl.ds(row0, CH)], fhi2_v)

            def _row(e, _):
                for q in range(HHD // 16):
                    fflat_v[pl.ds(e * HHD + q * 16, 16)] = (
                        fhi2_v[e, pl.ds(q * 16, 16)])
                return ()

            lax.fori_loop(0, CH, _row, ())
            pltpu.sync_copy(fflat_v,
                            dst_flat_hbm.at[pl.ds(row0 * HHD, CH * HHD)])
            return ()

        lax.fori_loop(0, ROWS_PER_TILE // CH, _piece, ())

    _zero_table()
    plsc.subcore_barrier()

    def _chunk_a(c, _):
        erow = wid * EPT + c * CH
        pltpu.sync_copy(srcf_hbm.at[pl.ds(erow, CH)], src_c)
        pltpu.sync_copy(dstf_hbm.at[pl.ds(erow, CH)], dst2_v.at[c])
        pltpu.async_copy(ft_hbm.at[src_c], ftr_v, sem).wait()
        pltpu.sync_copy(eftf_hbm.at[pl.ds(erow * HD, CH * HD)], eftf_v)
        pltpu.sync_copy(adropf_hbm.at[pl.ds(erow * H, CH * H)], adrop_v)

        def _edge(e, _):
            for h in range(H):
                sp = plsc.load_gather(adrop_v, [jnp.full((16,), e * H + h,
                                                         jnp.int32)])
                for q in range(2):
                    col = h * D + q * 16
                    off = pl.ds(e * HD + col, 16)
                    f = (ftr_v[e, pl.ds(col, 16)] + eftf_v[off]) * sp
                    if col < HHD:
                        flo_v[e, pl.ds(col, 16)] = f
                    else:
                        fflat_v[pl.ds(e * HHD + col - HHD, 16)] = f
                    eftf_v[off] = jnp.where(f > 0, f, jnp.exp(f) - 1.0)
            return ()

        lax.fori_loop(0, CH, _edge, ())
        pltpu.sync_copy(flo_v, acc_sp.at[dst2_v.at[c]], add=True)
        pltpu.sync_copy(fflat_v, fhi_hbm.at[pl.ds(erow * HHD, CH * HHD)])
        pltpu.sync_copy(eftf_v, ret2f_hbm.at[pl.ds(erow * HD, CH * HD)])
        return ()

    lax.fori_loop(0, NCH, _chunk_a, ())
    plsc.subcore_barrier()
    _dump_table(p2lo_hbm.at[cid])
    plsc.subcore_barrier()
    _zero_table()
    plsc.subcore_barrier()

    def _chunk_b(c, _):
        erow = wid * EPT + c * CH
        pltpu.sync_copy(fhi_hbm.at[pl.ds(erow * HHD, CH * HHD)], fflat_v)

        def _row(e, _):
            for q in range(HHD // 16):
                fhi2_v[e, pl.ds(q * 16, 16)] = (
                    fflat_v[pl.ds(e * HHD + q * 16, 16)])
            return ()

        lax.fori_loop(0, CH, _row, ())
        pltpu.sync_copy(fhi2_v, acc_sp.at[dst2_v.at[c]], add=True)
        return ()

    lax.fori_loop(0, NCH, _chunk_b, ())
    plsc.subcore_barrier()
    _dump_table(p2hi_hbm.at[cid])


def _run_k5(ft, eftf, adrop_flat, srcf, dstf):
    mesh = plsc.VectorSubcoreMesh(core_axis_name="c", subcore_axis_name="s")
    kern = functools.partial(
        pl.kernel,
        mesh=mesh,
        out_type=[
            jax.ShapeDtypeStruct((E * HD,), jnp.float32),
            jax.ShapeDtypeStruct((E * HHD,), jnp.float32),
            jax.ShapeDtypeStruct((2, NP * HHD), jnp.float32),
            jax.ShapeDtypeStruct((2, NP * HHD), jnp.float32),
        ],
        scratch_types=[
            pltpu.VMEM((CH,), jnp.int32),
            pltpu.VMEM((NCH, CH), jnp.int32),
            pltpu.VMEM((CH, HD), jnp.float32),
            pltpu.VMEM((CH * HD,), jnp.float32),
            pltpu.VMEM((CH, HHD), jnp.float32),
            pltpu.VMEM((CH, HHD), jnp.float32),
            pltpu.VMEM((CH * HHD,), jnp.float32),
            pltpu.VMEM((CH * H,), jnp.float32),
            pltpu.VMEM_SHARED((NP, HHD), jnp.float32),
            pltpu.SemaphoreType.DMA,
        ],
        compiler_params=pltpu.CompilerParams(needs_layout_passes=False,
                                             use_tc_tiling_on_sc=False),
    )(_k5_body)
    return kern(ft, eftf, adrop_flat, srcf, dstf)


def _k6_body(plo_ref, phi_ref, out_ref):
    s = jnp.concatenate([plo_ref[0] + plo_ref[1], phi_ref[0] + phi_ref[1]],
                        axis=1)
    out_ref[...] = jnp.where(s > 0, s, jnp.exp(s) - 1.0)


def _run_k6(p2lo, p2hi):
    blk = 2048
    return pl.pallas_call(
        _k6_body,
        grid=(NP // blk,),
        in_specs=[pl.BlockSpec((2, blk, HHD), lambda i: (0, i, 0)),
                  pl.BlockSpec((2, blk, HHD), lambda i: (0, i, 0))],
        out_specs=pl.BlockSpec((blk, HD), lambda i: (i, 0)),
        out_shape=jax.ShapeDtypeStruct((NP, HD), jnp.float32),
    )(p2lo, p2hi)


def kernel(node_feature, edge_feature, edge_index, W, attn_l, attn_r, attn_e):
    f32 = jnp.float32
    eye = jnp.eye(H, dtype=f32)
    s_l = (attn_l[..., 0][:, :, None] * eye[:, None, :]).reshape(HD, H)
    s_r = (attn_r[..., 0][:, :, None] * eye[:, None, :]).reshape(HD, H)
    s_e = (attn_e[..., 0][:, :, None] * eye[:, None, :]).reshape(HD, H)

    x_pad = jnp.pad(node_feature, ((0, NP - N), (0, 0)))

    ft, a1, a2, g1 = _run_k1(x_pad, W, s_l, s_r)
    eft, a3, g3 = _run_k2(edge_feature, W, s_e)

    g1v = g1.reshape(-1)[:16]
    g3v = g3.reshape(-1)[:16]
    srcf = edge_index[0].reshape(-1)
    dstf = edge_index[1].reshape(-1)
    z4 = jnp.zeros((NP * H,), f32)

    expa_flat, p_asum = _run_k3(a1.reshape(-1), a2.reshape(-1),
                                a3.reshape(-1), srcf, dstf, g1v, g3v, z4)
    recip = _run_k4(p_asum)
    adrop_flat = _run_k4b(expa_flat, recip.reshape(-1), dstf)
    ret2f, _, p2lo, p2hi = _run_k5(ft, eft.reshape(-1), adrop_flat, srcf,
                                   dstf)
    ret1 = _run_k6(p2lo.reshape(2, NP, HHD), p2hi.reshape(2, NP, HHD))[:N]
    return ret1, ret2f.reshape(E, HD)
